# baseline (speedup 1.0000x reference)
import functools

import jax
import jax.numpy as jnp
from jax import lax
from jax.experimental import pallas as pl
from jax.experimental.pallas import tpu as pltpu
from jax.experimental.pallas import tpu_sc as plsc

_S, _B, _D = 2048, 4, 1024
_S_TC = 1536
_S_SC = _S - _S_TC
_S_BLK = 256

_NC, _NS = 2, 16
_NW = _NC * _NS
_S_PER_W = _S_SC // _NW
_CH = 4
_N_CH = _S_PER_W // _CH
_L = 16


def _pe_add_kernel(x_ref, w_ref, o_ref):
    w = w_ref[...]
    for b in range(x_ref.shape[1]):
        o_ref[:, b, :] = x_ref[:, b, :] + w


def _tc_part(x, pos_embed_weight):
    grid = (_S_TC // _S_BLK,)
    return pl.pallas_call(
        _pe_add_kernel,
        grid=grid,
        in_specs=[
            pl.BlockSpec((_S_BLK, _B, _D), lambda i: (i, 0, 0)),
            pl.BlockSpec((_S_BLK, _D), lambda i: (i, 0)),
        ],
        out_specs=pl.BlockSpec((_S_BLK, _B, _D), lambda i: (i, 0, 0)),
        out_shape=jax.ShapeDtypeStruct((_S_TC, _B, _D), x.dtype),
    )(x, pos_embed_weight)


def _sc_body(x_hbm, w_hbm, out_hbm, x_v0, x_v1, w_v0, w_v1, o_v0, o_v1,
             six0, six1, siw0, siw1, so0, so1):
    xs, ws, os_ = (x_v0, x_v1), (w_v0, w_v1), (o_v0, o_v1)
    six, siw, so = (six0, six1), (siw0, siw1), (so0, so1)

    cid = lax.axis_index("c")
    sid = lax.axis_index("s")
    wid = sid * _NC + cid
    s_out = wid * _S_PER_W
    s_in = _S_TC + s_out

    def in_copies(c, b):
        s0 = s_in + c * _CH
        return (
            pltpu.make_async_copy(x_hbm.at[pl.ds(s0, _CH)], xs[b], six[b]),
            pltpu.make_async_copy(w_hbm.at[pl.ds(s0, _CH)], ws[b], siw[b]),
        )

    def out_copy(c, b):
        s0 = s_out + c * _CH
        return pltpu.make_async_copy(os_[b], out_hbm.at[pl.ds(s0, _CH)], so[b])

    def start_in(c, b):
        cx, cw = in_copies(c, b)
        cx.start()
        cw.start()

    start_in(0, 0)
    start_in(1, 1)

    def g_body(g, carry):
        for b in range(2):
            c = g * 2 + b

            @pl.when(c >= 2)
            def _():
                out_copy(c - 2, b).wait()

            cx, cw = in_copies(c, b)
            cx.wait()
            cw.wait()

            def j_body(j, carry2):
                dj = pl.ds(j * _L, _L)
                for s in range(_CH):
                    wv = ws[b][s, dj]
                    for bb in range(_B):
                        os_[b][s, bb, dj] = xs[b][s, bb, dj] + wv
                return carry2

            lax.fori_loop(0, _D // _L, j_body, 0)

            out_copy(c, b).start()

            @pl.when(c + 2 < _N_CH)
            def _():
                start_in(c + 2, b)
        return carry

    lax.fori_loop(0, _N_CH // 2, g_body, 0)

    out_copy(_N_CH - 2, 0).wait()
    out_copy(_N_CH - 1, 1).wait()


def _sc_part(x, pos_embed_weight):
    mesh = plsc.VectorSubcoreMesh(core_axis_name="c", subcore_axis_name="s")
    run = functools.partial(
        pl.kernel,
        mesh=mesh,
        out_type=jax.ShapeDtypeStruct((_S_SC, _B, _D), jnp.float32),
        scratch_types=[
            pltpu.VMEM((_CH, _B, _D), jnp.float32),
            pltpu.VMEM((_CH, _B, _D), jnp.float32),
            pltpu.VMEM((_CH, _D), jnp.float32),
            pltpu.VMEM((_CH, _D), jnp.float32),
            pltpu.VMEM((_CH, _B, _D), jnp.float32),
            pltpu.VMEM((_CH, _B, _D), jnp.float32),
            pltpu.SemaphoreType.DMA,
            pltpu.SemaphoreType.DMA,
            pltpu.SemaphoreType.DMA,
            pltpu.SemaphoreType.DMA,
            pltpu.SemaphoreType.DMA,
            pltpu.SemaphoreType.DMA,
        ],
    )(_sc_body)
    return run(x, pos_embed_weight)


def kernel(x, pos_embed_weight):
    out_tc = _tc_part(x, pos_embed_weight)
    out_sc = _sc_part(x, pos_embed_weight)
    return jnp.concatenate([out_tc, out_sc], axis=0)


# speedup vs baseline: 3.1435x; 3.1435x over previous
import jax
import jax.numpy as jnp
from jax.experimental import pallas as pl


_S_BLK = 512


def _pe_add_kernel(x_ref, w_ref, o_ref):
    w = w_ref[...]
    for b in range(x_ref.shape[1]):
        o_ref[:, b, :] = x_ref[:, b, :] + w


def kernel(x, pos_embed_weight):
    seq_len, batch, d_model = x.shape
    grid = (seq_len // _S_BLK,)
    return pl.pallas_call(
        _pe_add_kernel,
        grid=grid,
        in_specs=[
            pl.BlockSpec((_S_BLK, batch, d_model), lambda i: (i, 0, 0)),
            pl.BlockSpec((_S_BLK, d_model), lambda i: (i, 0)),
        ],
        out_specs=pl.BlockSpec((_S_BLK, batch, d_model), lambda i: (i, 0, 0)),
        out_shape=jax.ShapeDtypeStruct(x.shape, x.dtype),
    )(x, pos_embed_weight)
